# baseline (speedup 1.0000x reference)
import functools

import jax
import jax.numpy as jnp
import numpy as np
from jax import lax
from jax.experimental import pallas as pl
from jax.experimental.pallas import tpu as pltpu
from jax.experimental.pallas import tpu_sc as plsc

N = 10000
E = 320000
D = 128
NH = N // 2
NCLS = 64
SQRT_D = float(np.sqrt(128.0))

NCORE = 2
NSUB = 16
NWORK = NCORE * NSUB
EPW = E // NWORK
DEG_PAD = 10240
NVEC = EPW // 16

PCH = 128
QCH = 23
QCAP = QCH * PCH
DUMMY = NH
ACCN = 5008
PSTRIPE = 320
PLAST = ACCN - 15 * PSTRIPE
GSTRIPE = 320
GLAST = NH - 15 * GSTRIPE

_MESH = plsc.VectorSubcoreMesh(core_axis_name="c", subcore_axis_name="s")


@functools.partial(
    pl.kernel,
    mesh=_MESH,
    out_type=[jax.ShapeDtypeStruct((NWORK * QCAP,), jnp.int32)
              for _ in range(8)]
             + [jax.ShapeDtypeStruct((NWORK * DEG_PAD,), jnp.float32)],
    scratch_types=[
        pltpu.VMEM((EPW,), jnp.int32),
        pltpu.VMEM((EPW,), jnp.int32),
    ] + [pltpu.VMEM((QCAP + 16,), jnp.int32) for _ in range(8)]
      + [pltpu.VMEM((DEG_PAD,), jnp.float32)],
    compiler_params=pltpu.CompilerParams(needs_layout_passes=False),
)
def _sc_partition(src_hbm, dst_hbm,
                  os0, od0, os1, od1, os2, od2, os3, od3, odeg,
                  sv, dv, bs0, bd0, bs1, bd1, bs2, bd2, bs3, bd3, degb):
    c = lax.axis_index("c")
    s = lax.axis_index("s")
    wid = c * NSUB + s
    bs = (bs0, bs1, bs2, bs3)
    bd = (bd0, bd1, bd2, bd3)
    outs = ((os0, od0), (os1, od1), (os2, od2), (os3, od3))
    pltpu.sync_copy(src_hbm.at[pl.ds(wid * EPW, EPW)], sv)
    pltpu.sync_copy(dst_hbm.at[pl.ds(wid * EPW, EPW)], dv)

    def fill(i, carry):
        z = jnp.zeros((16,), jnp.int32)
        dmy = jnp.full((16,), DUMMY, jnp.int32)
        for q in range(4):
            bs[q][pl.ds(i * 16, 16)] = z
            bd[q][pl.ds(i * 16, 16)] = dmy
        return carry

    lax.fori_loop(0, QCAP // 16, fill, 0)

    def fillz(i, carry):
        degb[pl.ds(i * 16, 16)] = jnp.zeros((16,), jnp.float32)
        return carry

    lax.fori_loop(0, DEG_PAD // 16, fillz, 0)

    lane = lax.iota(jnp.int32, 16)
    zero16 = jnp.zeros((16,), jnp.int32)

    def body(i, carry):
        of0, of1, of2, of3 = carry
        sg = sv[pl.ds(i * 16, 16)]
        dg = dv[pl.ds(i * 16, 16)]
        plsc.addupdate_scatter(degb, [dg], jnp.ones((16,), jnp.float32))
        mdi = dg >= NH
        msi = sg >= NH
        key = jnp.where(mdi, 2, 0) + jnp.where(msi, 1, 0)
        srel = sg - jnp.where(msi, NH, 0)
        drel = dg - jnp.where(mdi, NH, 0)
        packed = srel * 8192 + drel
        _, vs = plsc.sort_key_val(key, packed)
        sgs = jnp.right_shift(vs, 13)
        dgs = jnp.bitwise_and(vs, 8191)
        k0 = plsc.all_reduce_population_count(key == 0)
        k01 = plsc.all_reduce_population_count(key <= 1)
        k012 = plsc.all_reduce_population_count(key <= 2)
        full = jnp.full((16,), 16, jnp.int32)
        starts = (zero16, k0, k01, k012)
        ends = (k0, k01, k012, full)
        offs = (of0, of1, of2, of3)
        trash = QCAP + lane
        news = []
        for q in range(4):
            inq = jnp.logical_and(lane >= starts[q], lane < ends[q])
            pos = jnp.where(inq, offs[q] + lane - starts[q], trash)
            plsc.store_scatter(bs[q], [pos], sgs)
            plsc.store_scatter(bd[q], [pos], dgs)
            news.append(offs[q] + (ends[q] - starts[q]))
        return tuple(news)

    lax.fori_loop(0, NVEC, body, (zero16, zero16, zero16, zero16))

    for q in range(4):
        pltpu.sync_copy(bs[q].at[pl.ds(0, QCAP)],
                        outs[q][0].at[pl.ds(wid * QCAP, QCAP)])
        pltpu.sync_copy(bd[q].at[pl.ds(0, QCAP)],
                        outs[q][1].at[pl.ds(wid * QCAP, QCAP)])
    pltpu.sync_copy(degb, odeg.at[pl.ds(wid * DEG_PAD, DEG_PAD)])


@functools.partial(
    pl.kernel,
    mesh=_MESH,
    out_type=jax.ShapeDtypeStruct((NCORE, ACCN, D), jnp.float32),
    scratch_types=[
        pltpu.VMEM((2, QCH, PCH), jnp.int32),
        pltpu.VMEM((2, QCH, PCH), jnp.int32),
        pltpu.VMEM_SHARED((NH, D), jnp.float32),
        pltpu.VMEM_SHARED((ACCN, D), jnp.float32),
        pltpu.VMEM((PCH, D), jnp.float32),
        pltpu.VMEM((PCH, D), jnp.float32),
        pltpu.SemaphoreType.DMA,
        pltpu.SemaphoreType.DMA,
        pltpu.SemaphoreType.DMA,
        pltpu.SemaphoreType.DMA,
    ],
)
def _sc_propagate(g_hbm, srcl_hbm, dstl_hbm, zeros_hbm, out_hbm,
                  sidx, didx, gbuf, acc, r0, r1, g0, g1, s0, s1):
    c = lax.axis_index("c")
    s = lax.axis_index("s")
    rows = (r0, r1)
    gsem = (g0, g1)
    ssem = (s0, s1)
    NCHT = 2 * QCH

    @pl.when(s < 15)
    def _():
        pltpu.sync_copy(zeros_hbm, acc.at[pl.ds(s * PSTRIPE, PSTRIPE)])

    @pl.when(s == 15)
    def _():
        pltpu.sync_copy(zeros_hbm.at[pl.ds(0, PLAST)],
                        acc.at[pl.ds(15 * PSTRIPE, PLAST)])

    for p in range(2):
        @pl.when(s < 15)
        def _(p=p):
            pltpu.sync_copy(g_hbm.at[pl.ds(p * NH + s * GSTRIPE, GSTRIPE)],
                            gbuf.at[pl.ds(s * GSTRIPE, GSTRIPE)])

        @pl.when(s == 15)
        def _(p=p):
            pltpu.sync_copy(g_hbm.at[pl.ds(p * NH + 15 * GSTRIPE, GLAST)],
                            gbuf.at[pl.ds(15 * GSTRIPE, GLAST)])

        pltpu.sync_copy(srcl_hbm.at[c].at[p].at[2 * s], sidx.at[0])
        pltpu.sync_copy(dstl_hbm.at[c].at[p].at[2 * s], didx.at[0])
        pltpu.sync_copy(srcl_hbm.at[c].at[p].at[2 * s + 1], sidx.at[1])
        pltpu.sync_copy(dstl_hbm.at[c].at[p].at[2 * s + 1], didx.at[1])
        plsc.subcore_barrier()

        def sidx_chunk(j):
            return sidx.at[j // QCH].at[j % QCH]

        def didx_chunk(j):
            return didx.at[j // QCH].at[j % QCH]

        pltpu.async_copy(gbuf.at[sidx_chunk(jnp.int32(0))], rows[0], gsem[0])

        def body(j, carry):
            for b in range(2):
                @pl.when(j % 2 == b)
                def _(b=b):
                    pb = (b + 1) % 2
                    pltpu.make_async_copy(gbuf.at[sidx_chunk(j)], rows[b],
                                          gsem[b]).wait()
                    pltpu.async_copy(rows[b], acc.at[didx_chunk(j)], ssem[b],
                                     add=True)

                    @pl.when(j + 1 < NCHT)
                    def _():
                        @pl.when(j >= 1)
                        def _():
                            pltpu.make_async_copy(rows[pb],
                                                  acc.at[didx_chunk(j)],
                                                  ssem[pb]).wait()
                        pltpu.async_copy(gbuf.at[sidx_chunk(j + 1)],
                                         rows[pb], gsem[pb])

            return carry

        lax.fori_loop(0, NCHT, body, 0)

        for b in range(2):
            pltpu.make_async_copy(rows[b], acc.at[didx.at[0].at[0]],
                                  ssem[b]).wait()

        plsc.subcore_barrier()

    @pl.when(s < 15)
    def _():
        pltpu.sync_copy(acc.at[pl.ds(s * PSTRIPE, PSTRIPE)],
                        out_hbm.at[c].at[pl.ds(s * PSTRIPE, PSTRIPE)])

    @pl.when(s == 15)
    def _():
        pltpu.sync_copy(acc.at[pl.ds(15 * PSTRIPE, PLAST)],
                        out_hbm.at[c].at[pl.ds(15 * PSTRIPE, PLAST)])


def _accsum(acc_ref):
    return jnp.concatenate([acc_ref[0, :NH], acc_ref[1, :NH]], axis=0)


def _tc_first_body(x_ref, w_ref, b_ref, deg_ref, g_ref, dinv_ref):
    deg = jnp.sum(deg_ref[...], axis=0)[:N, None] + 1.0
    dinv = lax.rsqrt(deg)
    z = jnp.dot(x_ref[...], w_ref[...], preferred_element_type=jnp.float32)
    g_ref[...] = dinv * (z + b_ref[...])
    dinv_ref[...] = dinv


def _tc_mid_body(acc_ref, g_ref, dinv_ref, w_ref, b_ref, h_ref, gout_ref):
    dinv = dinv_ref[...]
    h = jnp.maximum(dinv * (_accsum(acc_ref) + g_ref[...]), 0.0)
    h_ref[...] = h
    z = jnp.dot(h, w_ref[...], preferred_element_type=jnp.float32)
    gout_ref[...] = dinv * (z + b_ref[...])


def _tc_readout_body(acc_ref, g_ref, dinv_ref, h1_ref, h2_ref, wo_ref, bo_ref,
                     out_ref):
    dinv = dinv_ref[...]
    h3 = jnp.maximum(dinv * (_accsum(acc_ref) + g_ref[...]), 0.0)
    h1 = h1_ref[...]
    h2 = h2_ref[...]
    s1 = jnp.sum(h1 * h3, axis=1, keepdims=True) * (1.0 / SQRT_D)
    s2 = jnp.sum(h2 * h3, axis=1, keepdims=True) * (1.0 / SQRT_D)
    s3 = jnp.sum(h3 * h3, axis=1, keepdims=True) * (1.0 / SQRT_D)
    m = jnp.maximum(jnp.maximum(s1, s2), s3)
    e1 = jnp.exp(s1 - m)
    e2 = jnp.exp(s2 - m)
    e3 = jnp.exp(s3 - m)
    hsum = (e1 * h1 + e2 * h2 + e3 * h3) / (e1 + e2 + e3)
    out_ref[...] = (
        jnp.dot(hsum, wo_ref[...], preferred_element_type=jnp.float32)
        + bo_ref[...]
    )


def _tc_call(body, out_shapes, *args):
    return pl.pallas_call(
        body,
        out_shape=[jax.ShapeDtypeStruct(s, jnp.float32) for s in out_shapes],
    )(*args)


def kernel(x, edge_index, W0, b0, W1, b1, W2, b2, Wo, bo):
    src_flat = edge_index[0]
    dst_flat = edge_index[1]
    zeros = jnp.zeros((PSTRIPE, D), jnp.float32)

    s0, d0, s1, d1, s2, d2, s3, d3, degf = _sc_partition(src_flat, dst_flat)
    srcl = jnp.stack([s0, s1, s2, s3]).reshape(NCORE, 2, NWORK, QCH, PCH)
    dstl = jnp.stack([d0, d1, d2, d3]).reshape(NCORE, 2, NWORK, QCH, PCH)
    degp = degf.reshape(NWORK, DEG_PAD)

    g0, dinv = _tc_call(_tc_first_body, [(N, D), (N, 1)],
                        x, W0, b0.reshape(1, D), degp)

    acc0 = _sc_propagate(g0, srcl, dstl, zeros)
    h1, g1 = _tc_call(_tc_mid_body, [(N, D), (N, D)],
                      acc0, g0, dinv, W1, b1.reshape(1, D))

    acc1 = _sc_propagate(g1, srcl, dstl, zeros)
    h2, g2 = _tc_call(_tc_mid_body, [(N, D), (N, D)],
                      acc1, g1, dinv, W2, b2.reshape(1, D))

    acc2 = _sc_propagate(g2, srcl, dstl, zeros)
    (out,) = _tc_call(_tc_readout_body, [(N, NCLS)],
                      acc2, g2, dinv, h1, h2, Wo, bo.reshape(1, NCLS))
    return out
